# baseline (speedup 1.0000x reference)
import functools

import jax
import jax.numpy as jnp
from jax import lax
from jax.experimental import pallas as pl
from jax.experimental.pallas import tpu as pltpu
from jax.experimental.pallas import tpu_sc as plsc

N_EXPERTS = 64
TOP_K = 8
D_MODEL = 4096
TOKENS = 16384

BT = 512

NC = 2
NS = 16
L = 16
NW = NC * NS
TPW = TOKENS // NW


def _dense_block(x_ref, w_ref, b_ref, s_ref):
    s_ref[...] = jax.nn.sigmoid(
        jnp.dot(x_ref[...], w_ref[...], preferred_element_type=jnp.float32)
        + b_ref[...]
    )


def _tc_scores(x, W, b2):
    return pl.pallas_call(
        _dense_block,
        grid=(TOKENS // BT,),
        in_specs=[
            pl.BlockSpec((BT, D_MODEL), lambda i: (i, 0)),
            pl.BlockSpec((D_MODEL, N_EXPERTS), lambda i: (0, 0)),
            pl.BlockSpec((1, N_EXPERTS), lambda i: (0, 0)),
        ],
        out_specs=pl.BlockSpec((BT, N_EXPERTS), lambda i: (i, 0)),
        out_shape=jax.ShapeDtypeStruct((TOKENS, N_EXPERTS), jnp.float32),
    )(x, W, b2)


def _merge_tops(ka, ia, kb, ib):
    take_a = ka >= kb
    return jnp.where(take_a, ka, kb), jnp.where(take_a, ia, ib)


def _sc_topk_body(s_hbm, gs_hbm, gi_hbm, s_v, gs_v, gi_v):
    wid = lax.axis_index("s") * NC + lax.axis_index("c")
    base = wid * TPW
    pltpu.sync_copy(s_hbm.at[pl.ds(base, TPW)], s_v)

    iota = lax.iota(jnp.int32, L)
    lane_lt8 = iota < TOP_K

    @plsc.parallel_loop(0, TPW, step=1, unroll=4)
    def _token_loop(t):
        k0, i0 = plsc.sort_key_val(s_v[t, pl.ds(0, L)], iota, descending=True)
        k1, i1 = plsc.sort_key_val(s_v[t, pl.ds(L, L)], iota + L)
        k2, i2 = plsc.sort_key_val(s_v[t, pl.ds(2 * L, L)], iota + 2 * L, descending=True)
        k3, i3 = plsc.sort_key_val(s_v[t, pl.ds(3 * L, L)], iota + 3 * L)
        ek, ei = _merge_tops(k0, i0, k1, i1)
        fk, fi = _merge_tops(k2, i2, k3, i3)
        ek, ei = plsc.sort_key_val(ek, ei, descending=True)
        fk, fi = plsc.sort_key_val(fk, fi)
        gk, gi = _merge_tops(ek, ei, fk, fi)
        gk, gi = plsc.sort_key_val(gk, gi, descending=True)
        total = jnp.sum(jnp.where(lane_lt8, gk, 0.0))
        gs_v[t, :] = gk / total
        gi_v[t, :] = gi

    pltpu.sync_copy(gs_v, gs_hbm.at[pl.ds(base, TPW)])
    pltpu.sync_copy(gi_v, gi_hbm.at[pl.ds(base, TPW)])


_sc_topk = pl.kernel(
    _sc_topk_body,
    out_type=[
        jax.ShapeDtypeStruct((TOKENS, L), jnp.float32),
        jax.ShapeDtypeStruct((TOKENS, L), jnp.int32),
    ],
    mesh=plsc.VectorSubcoreMesh(
        core_axis_name="c", subcore_axis_name="s", num_cores=NC, num_subcores=NS
    ),
    scratch_types=[
        pltpu.VMEM((TPW, N_EXPERTS), jnp.float32),
        pltpu.VMEM((TPW, L), jnp.float32),
        pltpu.VMEM((TPW, L), jnp.int32),
    ],
    compiler_params=pltpu.CompilerParams(
        needs_layout_passes=False, use_tc_tiling_on_sc=False
    ),
)


@jax.jit
def kernel(x, W, b):
    s = _tc_scores(x, W, b.reshape(1, N_EXPERTS))
    gs16, gi16 = _sc_topk(s)
    return (gs16[:, :TOP_K], gi16[:, :TOP_K], s)


# speedup vs baseline: 1.0077x; 1.0077x over previous
import jax
import jax.numpy as jnp
from jax import lax
from jax.experimental import pallas as pl
from jax.experimental.pallas import tpu as pltpu
from jax.experimental.pallas import tpu_sc as plsc

N_EXPERTS = 64
TOP_K = 8
D_MODEL = 4096
TOKENS = 16384

BT = 512

NC = 2
NS = 16
L = 16
NW = NC * NS
TPW = TOKENS // NW


def _dense_block(x_ref, w_ref, b_ref, s_ref):
    s_ref[...] = jax.nn.sigmoid(
        jnp.dot(x_ref[...], w_ref[...], preferred_element_type=jnp.float32)
        + b_ref[...]
    )


def _tc_scores(x, W, b2):
    return pl.pallas_call(
        _dense_block,
        grid=(TOKENS // BT,),
        in_specs=[
            pl.BlockSpec((BT, D_MODEL), lambda i: (i, 0)),
            pl.BlockSpec((D_MODEL, N_EXPERTS), lambda i: (0, 0)),
            pl.BlockSpec((1, N_EXPERTS), lambda i: (0, 0)),
        ],
        out_specs=pl.BlockSpec((BT, N_EXPERTS), lambda i: (i, 0)),
        out_shape=jax.ShapeDtypeStruct((TOKENS, N_EXPERTS), jnp.float32),
    )(x, W, b2)


def _merge_tops(ka, ia, kb, ib):
    take_a = ka >= kb
    return jnp.where(take_a, ka, kb), jnp.where(take_a, ia, ib)


def _sc_topk_body(s_hbm, gs_hbm, gi_hbm, s_v, gs_v, gi_v):
    wid = lax.axis_index("s") * NC + lax.axis_index("c")
    base = wid * TPW
    pltpu.sync_copy(s_hbm.at[pl.ds(base, TPW)], s_v)

    iota = lax.iota(jnp.int32, L)
    lane_lt8 = iota < TOP_K

    @plsc.parallel_loop(0, TPW, step=1, unroll=4)
    def _token_loop(t):
        k0, i0 = plsc.sort_key_val(s_v[t, pl.ds(0, L)], iota, descending=True)
        k1, i1 = plsc.sort_key_val(s_v[t, pl.ds(L, L)], iota + L)
        k2, i2 = plsc.sort_key_val(s_v[t, pl.ds(2 * L, L)], iota + 2 * L, descending=True)
        k3, i3 = plsc.sort_key_val(s_v[t, pl.ds(3 * L, L)], iota + 3 * L)
        ek, ei = _merge_tops(k0, i0, k1, i1)
        fk, fi = _merge_tops(k2, i2, k3, i3)
        ek, ei = plsc.sort_key_val(ek, ei, descending=True)
        fk, fi = plsc.sort_key_val(fk, fi)
        gk, gi = _merge_tops(ek, ei, fk, fi)
        gk, gi = plsc.sort_key_val(gk, gi, descending=True)
        total = jnp.sum(jnp.where(lane_lt8, gk, 0.0))
        gs_v[t, :] = gk / total
        gi_v[t, :] = gi

    pltpu.sync_copy(gs_v, gs_hbm.at[pl.ds(base, TPW)])
    pltpu.sync_copy(gi_v, gi_hbm.at[pl.ds(base, TPW)])


_sc_topk = pl.kernel(
    _sc_topk_body,
    out_type=[
        jax.ShapeDtypeStruct((TOKENS, L), jnp.float32),
        jax.ShapeDtypeStruct((TOKENS, L), jnp.int32),
    ],
    mesh=plsc.VectorSubcoreMesh(
        core_axis_name="c", subcore_axis_name="s", num_cores=NC, num_subcores=NS
    ),
    scratch_types=[
        pltpu.VMEM((TPW, N_EXPERTS), jnp.float32),
        pltpu.VMEM((TPW, L), jnp.float32),
        pltpu.VMEM((TPW, L), jnp.int32),
    ],
    compiler_params=pltpu.CompilerParams(
        needs_layout_passes=False, use_tc_tiling_on_sc=False
    ),
)


@jax.jit
def kernel(x, W, b):
    s = _tc_scores(x, W, b.reshape(1, N_EXPERTS))
    gs16, gi16 = _sc_topk(s)
    return (gs16[:, :TOP_K], gi16[:, :TOP_K], s)
